# Initial kernel scaffold; baseline (speedup 1.0000x reference)
import jax
import jax.numpy as jnp
from jax.experimental import pallas as pl


def kernel(x, edge_index_list, W1, b1, W2, b2, Wc):
    raise NotImplementedError("write your pallas kernel here")


# speedup vs baseline: 29.3554x; 29.3554x over previous
import functools

import jax
import jax.numpy as jnp
from jax import lax
from jax.experimental import pallas as pl
from jax.experimental.pallas import tpu as pltpu
from jax.experimental.pallas import tpu_sc as plsc

NUM_NODES = 50000
SEQ = 12
HID = 64
NC = 2
NS = 16
NW = NC * NS
CHUNK = 128
CPW = 208
EPW = CHUNK * CPW
EPAD = EPW * NW
NPAD = 51200
D = 16
ROWS_PS = NPAD // NS

_mesh = plsc.VectorSubcoreMesh(core_axis_name="c", subcore_axis_name="s")


@functools.partial(
    pl.kernel,
    mesh=_mesh,
    out_type=jax.ShapeDtypeStruct((NC, NPAD, D), jnp.float32),
    compiler_params=pltpu.CompilerParams(use_tc_tiling_on_sc=False),
    scratch_types=[
        pltpu.VMEM((CPW, CHUNK), jnp.int32),
        pltpu.VMEM((CHUNK, D), jnp.float32),
        pltpu.VMEM_SHARED((NPAD, D), jnp.float32),
        pltpu.SemaphoreType.DMA,
    ],
)
def _deg_kernel(dst_hbm, ones_hbm, zeros_hbm, out_hbm, idx_v, ones_v, acc, sem):
    cid = lax.axis_index("c")
    sid = lax.axis_index("s")
    wid = sid * NC + cid
    pltpu.sync_copy(ones_hbm, ones_v)
    pltpu.sync_copy(zeros_hbm.at[pl.ds(sid * ROWS_PS, ROWS_PS)],
                    acc.at[pl.ds(sid * ROWS_PS, ROWS_PS)])
    pltpu.sync_copy(dst_hbm.at[pl.ds(wid * CPW, CPW)], idx_v)
    plsc.subcore_barrier()

    def body(j, carry):
        pltpu.sync_copy(ones_v, acc.at[idx_v.at[j]], add=True)
        return carry

    lax.fori_loop(0, CPW, body, 0)
    plsc.subcore_barrier()
    pltpu.sync_copy(acc.at[pl.ds(sid * ROWS_PS, ROWS_PS)],
                    out_hbm.at[cid, pl.ds(sid * ROWS_PS, ROWS_PS)])


@functools.partial(
    pl.kernel,
    mesh=_mesh,
    out_type=jax.ShapeDtypeStruct((NC, NPAD, D), jnp.float32),
    compiler_params=pltpu.CompilerParams(use_tc_tiling_on_sc=False),
    scratch_types=[
        pltpu.VMEM((CPW, CHUNK), jnp.int32),
        pltpu.VMEM((CPW, CHUNK), jnp.int32),
        pltpu.VMEM((CHUNK, D), jnp.float32),
        pltpu.VMEM_SHARED((NPAD, D), jnp.float32),
        pltpu.SemaphoreType.DMA,
    ],
)
def _agg_kernel(y_hbm, src_hbm, dst_hbm, zeros_hbm, out_hbm,
                src_v, dst_v, rows_v, acc, sem):
    cid = lax.axis_index("c")
    sid = lax.axis_index("s")
    wid = sid * NC + cid
    pltpu.sync_copy(zeros_hbm.at[pl.ds(sid * ROWS_PS, ROWS_PS)],
                    acc.at[pl.ds(sid * ROWS_PS, ROWS_PS)])
    pltpu.sync_copy(src_hbm.at[pl.ds(wid * CPW, CPW)], src_v)
    pltpu.sync_copy(dst_hbm.at[pl.ds(wid * CPW, CPW)], dst_v)
    plsc.subcore_barrier()

    def body(j, carry):
        pltpu.async_copy(y_hbm.at[src_v.at[j]], rows_v, sem).wait()
        pltpu.sync_copy(rows_v, acc.at[dst_v.at[j]], add=True)
        return carry

    lax.fori_loop(0, CPW, body, 0)
    plsc.subcore_barrier()
    pltpu.sync_copy(acc.at[pl.ds(sid * ROWS_PS, ROWS_PS)],
                    out_hbm.at[cid, pl.ds(sid * ROWS_PS, ROWS_PS)])


def _pad_edges(idx, pad_e, fill):
    flat = jnp.concatenate(
        [idx, jnp.full((pad_e,), fill, jnp.int32)])
    return flat.reshape(EPAD // CHUNK, CHUNK)


def kernel(x, edge_index_list, W1, b1, W2, b2, Wc):
    batch = x.shape[0]
    n_edges = edge_index_list.shape[2]
    loop = jnp.arange(NUM_NODES, dtype=jnp.int32)
    pad_e = EPAD - (n_edges + NUM_NODES)
    zeros_init = jnp.zeros((NPAD, D), jnp.float32)
    ones_rows = jnp.ones((CHUNK, D), jnp.float32)

    outs = []
    for bi in range(batch):
        src = _pad_edges(
            jnp.concatenate([edge_index_list[bi, 0], loop]), pad_e, NUM_NODES)
        dst = _pad_edges(
            jnp.concatenate([edge_index_list[bi, 1], loop]), pad_e, NUM_NODES)

        deg_part = _deg_kernel(dst, ones_rows, zeros_init)
        deg = deg_part[0, :NUM_NODES, 0] + deg_part[1, :NUM_NODES, 0]
        dinv = lax.rsqrt(deg)

        xb = x[bi].T
        y1 = jnp.zeros((NPAD, D), jnp.float32).at[:NUM_NODES, :SEQ].set(
            xb * dinv[:, None])
        s1 = _agg_kernel(y1, src, dst, zeros_init)
        a1 = (s1[0, :NUM_NODES, :SEQ] + s1[1, :NUM_NODES, :SEQ]) * dinv[:, None]
        h1 = jax.nn.relu(a1 @ W1 + b1)
        g = h1 @ W2
        y2 = jnp.zeros((NPAD, D), jnp.float32).at[:NUM_NODES, :SEQ].set(
            g * dinv[:, None])
        s2 = _agg_kernel(y2, src, dst, zeros_init)
        ht = (s2[0, :NUM_NODES, :SEQ] + s2[1, :NUM_NODES, :SEQ]) * dinv[:, None] + b2

        out_t = (Wc[:, :, 0] @ jnp.roll(ht, 1, axis=1)
                 + Wc[:, :, 1] @ ht
                 + Wc[:, :, 2] @ jnp.roll(ht, -1, axis=1))
        outs.append(out_t.T[None])
    return jnp.concatenate(outs, axis=0)


# speedup vs baseline: 29.4793x; 1.0042x over previous
import functools

import jax
import jax.numpy as jnp
from jax import lax
from jax.experimental import pallas as pl
from jax.experimental.pallas import tpu as pltpu
from jax.experimental.pallas import tpu_sc as plsc

NUM_NODES = 50000
SEQ = 12
HID = 64
NC = 2
NS = 16
NW = NC * NS
CHUNK = 128
CPW = 208
EPW = CHUNK * CPW
EPAD = EPW * NW
NPAD = 51200
D = 16
ROWS_PS = NPAD // NS

_mesh = plsc.VectorSubcoreMesh(core_axis_name="c", subcore_axis_name="s")


@functools.partial(
    pl.kernel,
    mesh=_mesh,
    out_type=jax.ShapeDtypeStruct((NC, NPAD, D), jnp.float32),
    compiler_params=pltpu.CompilerParams(use_tc_tiling_on_sc=False),
    scratch_types=[
        pltpu.VMEM((CPW, CHUNK), jnp.int32),
        pltpu.VMEM((CHUNK, D), jnp.float32),
        pltpu.VMEM_SHARED((NPAD, D), jnp.float32),
        pltpu.SemaphoreType.DMA,
    ],
)
def _deg_kernel(dst_hbm, ones_hbm, zeros_hbm, out_hbm, idx_v, ones_v, acc, sem):
    cid = lax.axis_index("c")
    sid = lax.axis_index("s")
    wid = sid * NC + cid
    pltpu.sync_copy(ones_hbm, ones_v)
    pltpu.sync_copy(zeros_hbm.at[pl.ds(sid * ROWS_PS, ROWS_PS)],
                    acc.at[pl.ds(sid * ROWS_PS, ROWS_PS)])
    pltpu.sync_copy(dst_hbm.at[pl.ds(wid * CPW, CPW)], idx_v)
    plsc.subcore_barrier()

    def body(j, carry):
        pltpu.sync_copy(ones_v, acc.at[idx_v.at[j]], add=True)
        return carry

    lax.fori_loop(0, CPW, body, 0)
    plsc.subcore_barrier()
    pltpu.sync_copy(acc.at[pl.ds(sid * ROWS_PS, ROWS_PS)],
                    out_hbm.at[cid, pl.ds(sid * ROWS_PS, ROWS_PS)])


@functools.partial(
    pl.kernel,
    mesh=_mesh,
    out_type=jax.ShapeDtypeStruct((NC, NPAD, D), jnp.float32),
    compiler_params=pltpu.CompilerParams(use_tc_tiling_on_sc=False),
    scratch_types=[
        pltpu.VMEM((CPW, CHUNK), jnp.int32),
        pltpu.VMEM((CPW, CHUNK), jnp.int32),
        pltpu.VMEM((CHUNK, D), jnp.float32),
        pltpu.VMEM_SHARED((NPAD, D), jnp.float32),
        pltpu.SemaphoreType.DMA,
    ],
)
def _agg_kernel(y_hbm, src_hbm, dst_hbm, zeros_hbm, out_hbm,
                src_v, dst_v, rows_v, acc, sem):
    cid = lax.axis_index("c")
    sid = lax.axis_index("s")
    wid = sid * NC + cid
    pltpu.sync_copy(zeros_hbm.at[pl.ds(sid * ROWS_PS, ROWS_PS)],
                    acc.at[pl.ds(sid * ROWS_PS, ROWS_PS)])
    pltpu.sync_copy(src_hbm.at[pl.ds(wid * CPW, CPW)], src_v)
    pltpu.sync_copy(dst_hbm.at[pl.ds(wid * CPW, CPW)], dst_v)
    plsc.subcore_barrier()

    def body(j, carry):
        pltpu.async_copy(y_hbm.at[src_v.at[j]], rows_v, sem).wait()
        pltpu.sync_copy(rows_v, acc.at[dst_v.at[j]], add=True)
        return carry

    lax.fori_loop(0, CPW, body, 0)
    plsc.subcore_barrier()
    pltpu.sync_copy(acc.at[pl.ds(sid * ROWS_PS, ROWS_PS)],
                    out_hbm.at[cid, pl.ds(sid * ROWS_PS, ROWS_PS)])


BLK_MID = 2048
NBLK_MID = NPAD // BLK_MID
BLK_CONV = 2000
NBLK_CONV = NUM_NODES // BLK_CONV


def _mid_body(s1_ref, deg_ref, w1_ref, b1_ref, w2_ref, y2_ref):
    dinv = lax.rsqrt(jnp.maximum(deg_ref[:, :1], 1.0))
    a1 = s1_ref[:, :SEQ] * dinv
    h1 = jnp.maximum(jnp.dot(a1, w1_ref[...],
                             preferred_element_type=jnp.float32) + b1_ref[...], 0.0)
    g = jnp.dot(h1, w2_ref[...], preferred_element_type=jnp.float32)
    y2_ref[:, :SEQ] = g * dinv
    y2_ref[:, SEQ:] = jnp.zeros((BLK_MID, D - SEQ), jnp.float32)


_mid_call = pl.pallas_call(
    _mid_body,
    grid=(NBLK_MID,),
    in_specs=[
        pl.BlockSpec((BLK_MID, D), lambda i: (i, 0)),
        pl.BlockSpec((BLK_MID, D), lambda i: (i, 0)),
        pl.BlockSpec((SEQ, HID), lambda i: (0, 0)),
        pl.BlockSpec((1, HID), lambda i: (0, 0)),
        pl.BlockSpec((HID, SEQ), lambda i: (0, 0)),
    ],
    out_specs=pl.BlockSpec((BLK_MID, D), lambda i: (i, 0)),
    out_shape=jax.ShapeDtypeStruct((NPAD, D), jnp.float32),
)


def _conv_body(s2a_ref, s2b_ref, dega_ref, degb_ref, b2_ref,
               w0_ref, w1_ref, w2_ref, out_ref):
    dinva = lax.rsqrt(jnp.maximum(dega_ref[:, :1], 1.0))
    dinvb = lax.rsqrt(jnp.maximum(degb_ref[:, :1], 1.0))
    hta = s2a_ref[:, :SEQ] * dinva + b2_ref[...]
    htb = s2b_ref[:, :SEQ] * dinvb + b2_ref[...]
    ht = jnp.concatenate([hta, htb], axis=1)
    dn = (((0,), (0,)), ((), ()))
    m0 = lax.dot_general(ht, w0_ref[...], dn,
                         preferred_element_type=jnp.float32)
    m1 = lax.dot_general(ht, w1_ref[...], dn,
                         preferred_element_type=jnp.float32)
    m2 = lax.dot_general(ht, w2_ref[...], dn,
                         preferred_element_type=jnp.float32)

    def roll1(a):
        return jnp.concatenate([a[-1:], a[:-1]], axis=0)

    def rollm1(a):
        return jnp.concatenate([a[1:], a[:1]], axis=0)

    acc_halves = []
    for h in range(2):
        sl = slice(h * SEQ, (h + 1) * SEQ)
        acc_halves.append(roll1(m0[sl]) + m1[sl] + rollm1(m2[sl]))
    acc = jnp.concatenate(acc_halves, axis=0)

    @pl.when(pl.program_id(0) == 0)
    def _init():
        out_ref[...] = jnp.zeros_like(out_ref)

    out_ref[...] += acc


_conv_call = pl.pallas_call(
    _conv_body,
    grid=(NBLK_CONV,),
    in_specs=[
        pl.BlockSpec((BLK_CONV, D), lambda i: (i, 0)),
        pl.BlockSpec((BLK_CONV, D), lambda i: (i, 0)),
        pl.BlockSpec((BLK_CONV, D), lambda i: (i, 0)),
        pl.BlockSpec((BLK_CONV, D), lambda i: (i, 0)),
        pl.BlockSpec((1, SEQ), lambda i: (0, 0)),
        pl.BlockSpec((BLK_CONV, HID), lambda i: (i, 0)),
        pl.BlockSpec((BLK_CONV, HID), lambda i: (i, 0)),
        pl.BlockSpec((BLK_CONV, HID), lambda i: (i, 0)),
    ],
    out_specs=pl.BlockSpec((2 * SEQ, HID), lambda i: (0, 0)),
    out_shape=jax.ShapeDtypeStruct((2 * SEQ, HID), jnp.float32),
)


def _pad_edges(idx, pad_e, fill):
    flat = jnp.concatenate(
        [idx, jnp.full((pad_e,), fill, jnp.int32)])
    return flat.reshape(EPAD // CHUNK, CHUNK)


def kernel(x, edge_index_list, W1, b1, W2, b2, Wc):
    batch = x.shape[0]
    n_edges = edge_index_list.shape[2]
    loop = jnp.arange(NUM_NODES, dtype=jnp.int32)
    pad_e = EPAD - (n_edges + NUM_NODES)
    zeros_init = jnp.zeros((NPAD, D), jnp.float32)
    ones_rows = jnp.ones((CHUNK, D), jnp.float32)

    degsums = []
    ssum2s = []
    for bi in range(batch):
        src = _pad_edges(
            jnp.concatenate([edge_index_list[bi, 0], loop]), pad_e, NUM_NODES)
        dst = _pad_edges(
            jnp.concatenate([edge_index_list[bi, 1], loop]), pad_e, NUM_NODES)

        deg_part = _deg_kernel(dst, ones_rows, zeros_init)
        degsum = deg_part[0] + deg_part[1]
        dinv = lax.rsqrt(degsum[:NUM_NODES, :1])

        xb = x[bi].T
        y1 = jnp.zeros((NPAD, D), jnp.float32).at[:NUM_NODES, :SEQ].set(xb * dinv)
        s1 = _agg_kernel(y1, src, dst, zeros_init)
        y2 = _mid_call(s1[0] + s1[1], degsum, W1, b1.reshape(1, HID), W2)
        s2 = _agg_kernel(y2, src, dst, zeros_init)

        degsums.append(degsum[:NUM_NODES])
        ssum2s.append((s2[0] + s2[1])[:NUM_NODES])

    out_flat = _conv_call(ssum2s[0], ssum2s[1], degsums[0], degsums[1],
                          b2.reshape(1, SEQ), Wc[:, :, 0].T, Wc[:, :, 1].T,
                          Wc[:, :, 2].T)
    return jnp.stack([out_flat[:SEQ], out_flat[SEQ:]], axis=0)


# speedup vs baseline: 37.9706x; 1.2880x over previous
import functools

import jax
import jax.numpy as jnp
from jax import lax
from jax.experimental import pallas as pl
from jax.experimental.pallas import tpu as pltpu
from jax.experimental.pallas import tpu_sc as plsc

NUM_NODES = 50000
SEQ = 12
HID = 64
NC = 2
NS = 16
NW = NC * NS
CHUNK = 128
CPW = 208
EPW = CHUNK * CPW
EPAD = EPW * NW
NPAD = 51200
D = 16
ROWS_PS = NPAD // NS

_mesh = plsc.VectorSubcoreMesh(core_axis_name="c", subcore_axis_name="s")


@functools.partial(
    pl.kernel,
    mesh=_mesh,
    out_type=jax.ShapeDtypeStruct((NC, NPAD, D), jnp.float32),
    compiler_params=pltpu.CompilerParams(use_tc_tiling_on_sc=False),
    scratch_types=[
        pltpu.VMEM((CPW, CHUNK), jnp.int32),
        pltpu.VMEM((CHUNK, D), jnp.float32),
        pltpu.VMEM_SHARED((NPAD, D), jnp.float32),
        pltpu.SemaphoreType.DMA,
    ],
)
def _deg_kernel(dst_hbm, ones_hbm, zeros_hbm, out_hbm, idx_v, ones_v, acc, sem):
    cid = lax.axis_index("c")
    sid = lax.axis_index("s")
    wid = sid * NC + cid
    pltpu.sync_copy(ones_hbm, ones_v)
    pltpu.sync_copy(zeros_hbm.at[pl.ds(sid * ROWS_PS, ROWS_PS)],
                    acc.at[pl.ds(sid * ROWS_PS, ROWS_PS)])
    pltpu.sync_copy(dst_hbm.at[pl.ds(wid * CPW, CPW)], idx_v)
    plsc.subcore_barrier()

    def body(j, carry):
        pltpu.sync_copy(ones_v, acc.at[idx_v.at[j]], add=True)
        return carry

    lax.fori_loop(0, CPW, body, 0)
    plsc.subcore_barrier()
    pltpu.sync_copy(acc.at[pl.ds(sid * ROWS_PS, ROWS_PS)],
                    out_hbm.at[cid, pl.ds(sid * ROWS_PS, ROWS_PS)])


@functools.partial(
    pl.kernel,
    mesh=_mesh,
    out_type=jax.ShapeDtypeStruct((NC, NPAD, D), jnp.float32),
    compiler_params=pltpu.CompilerParams(use_tc_tiling_on_sc=False),
    scratch_types=[
        pltpu.VMEM((CPW, CHUNK), jnp.int32),
        pltpu.VMEM((CPW, CHUNK), jnp.int32),
        pltpu.VMEM((4, CHUNK, D), jnp.float32),
        pltpu.VMEM_SHARED((NPAD, D), jnp.float32),
        pltpu.SemaphoreType.DMA,
        pltpu.SemaphoreType.DMA,
        pltpu.SemaphoreType.DMA,
        pltpu.SemaphoreType.DMA,
    ],
)
def _agg_kernel(y_hbm, src_hbm, dst_hbm, zeros_hbm, out_hbm,
                src_v, dst_v, rows_v, acc, s0, s1, s2, s3):
    cid = lax.axis_index("c")
    sid = lax.axis_index("s")
    wid = sid * NC + cid
    sems = (s0, s1, s2, s3)
    pltpu.sync_copy(zeros_hbm.at[pl.ds(sid * ROWS_PS, ROWS_PS)],
                    acc.at[pl.ds(sid * ROWS_PS, ROWS_PS)])
    pltpu.sync_copy(src_hbm.at[pl.ds(wid * CPW, CPW)], src_v)
    pltpu.sync_copy(dst_hbm.at[pl.ds(wid * CPW, CPW)], dst_v)
    plsc.subcore_barrier()

    def body(g, carry):
        base = g * 4
        handles = [
            pltpu.async_copy(y_hbm.at[src_v.at[base + b]], rows_v.at[b], sems[b])
            for b in range(4)
        ]
        for b in range(4):
            handles[b].wait()
            pltpu.sync_copy(rows_v.at[b], acc.at[dst_v.at[base + b]], add=True)
        return carry

    lax.fori_loop(0, CPW // 4, body, 0)
    plsc.subcore_barrier()
    pltpu.sync_copy(acc.at[pl.ds(sid * ROWS_PS, ROWS_PS)],
                    out_hbm.at[cid, pl.ds(sid * ROWS_PS, ROWS_PS)])


BLK_MID = 2048
NBLK_MID = NPAD // BLK_MID
BLK_CONV = 2000
NBLK_CONV = NUM_NODES // BLK_CONV


def _mid_body(s1_ref, deg_ref, w1_ref, b1_ref, w2_ref, y2_ref):
    dinv = lax.rsqrt(jnp.maximum(deg_ref[:, :1], 1.0))
    a1 = s1_ref[:, :SEQ] * dinv
    h1 = jnp.maximum(jnp.dot(a1, w1_ref[...],
                             preferred_element_type=jnp.float32) + b1_ref[...], 0.0)
    g = jnp.dot(h1, w2_ref[...], preferred_element_type=jnp.float32)
    y2_ref[:, :SEQ] = g * dinv
    y2_ref[:, SEQ:] = jnp.zeros((BLK_MID, D - SEQ), jnp.float32)


_mid_call = pl.pallas_call(
    _mid_body,
    grid=(NBLK_MID,),
    in_specs=[
        pl.BlockSpec((BLK_MID, D), lambda i: (i, 0)),
        pl.BlockSpec((BLK_MID, D), lambda i: (i, 0)),
        pl.BlockSpec((SEQ, HID), lambda i: (0, 0)),
        pl.BlockSpec((1, HID), lambda i: (0, 0)),
        pl.BlockSpec((HID, SEQ), lambda i: (0, 0)),
    ],
    out_specs=pl.BlockSpec((BLK_MID, D), lambda i: (i, 0)),
    out_shape=jax.ShapeDtypeStruct((NPAD, D), jnp.float32),
)


def _conv_body(s2a_ref, s2b_ref, dega_ref, degb_ref, b2_ref,
               w0_ref, w1_ref, w2_ref, out_ref):
    dinva = lax.rsqrt(jnp.maximum(dega_ref[:, :1], 1.0))
    dinvb = lax.rsqrt(jnp.maximum(degb_ref[:, :1], 1.0))
    hta = s2a_ref[:, :SEQ] * dinva + b2_ref[...]
    htb = s2b_ref[:, :SEQ] * dinvb + b2_ref[...]
    ht = jnp.concatenate([hta, htb], axis=1)
    dn = (((0,), (0,)), ((), ()))
    m0 = lax.dot_general(ht, w0_ref[...], dn,
                         preferred_element_type=jnp.float32)
    m1 = lax.dot_general(ht, w1_ref[...], dn,
                         preferred_element_type=jnp.float32)
    m2 = lax.dot_general(ht, w2_ref[...], dn,
                         preferred_element_type=jnp.float32)

    def roll1(a):
        return jnp.concatenate([a[-1:], a[:-1]], axis=0)

    def rollm1(a):
        return jnp.concatenate([a[1:], a[:1]], axis=0)

    acc_halves = []
    for h in range(2):
        sl = slice(h * SEQ, (h + 1) * SEQ)
        acc_halves.append(roll1(m0[sl]) + m1[sl] + rollm1(m2[sl]))
    acc = jnp.concatenate(acc_halves, axis=0)

    @pl.when(pl.program_id(0) == 0)
    def _init():
        out_ref[...] = jnp.zeros_like(out_ref)

    out_ref[...] += acc


_conv_call = pl.pallas_call(
    _conv_body,
    grid=(NBLK_CONV,),
    in_specs=[
        pl.BlockSpec((BLK_CONV, D), lambda i: (i, 0)),
        pl.BlockSpec((BLK_CONV, D), lambda i: (i, 0)),
        pl.BlockSpec((BLK_CONV, D), lambda i: (i, 0)),
        pl.BlockSpec((BLK_CONV, D), lambda i: (i, 0)),
        pl.BlockSpec((1, SEQ), lambda i: (0, 0)),
        pl.BlockSpec((BLK_CONV, HID), lambda i: (i, 0)),
        pl.BlockSpec((BLK_CONV, HID), lambda i: (i, 0)),
        pl.BlockSpec((BLK_CONV, HID), lambda i: (i, 0)),
    ],
    out_specs=pl.BlockSpec((2 * SEQ, HID), lambda i: (0, 0)),
    out_shape=jax.ShapeDtypeStruct((2 * SEQ, HID), jnp.float32),
)


def _pad_edges(idx, pad_e, fill):
    flat = jnp.concatenate(
        [idx, jnp.full((pad_e,), fill, jnp.int32)])
    return flat.reshape(EPAD // CHUNK, CHUNK)


def kernel(x, edge_index_list, W1, b1, W2, b2, Wc):
    batch = x.shape[0]
    n_edges = edge_index_list.shape[2]
    loop = jnp.arange(NUM_NODES, dtype=jnp.int32)
    pad_e = EPAD - (n_edges + NUM_NODES)
    zeros_init = jnp.zeros((NPAD, D), jnp.float32)
    ones_rows = jnp.ones((CHUNK, D), jnp.float32)

    degsums = []
    ssum2s = []
    for bi in range(batch):
        src = _pad_edges(
            jnp.concatenate([edge_index_list[bi, 0], loop]), pad_e, NUM_NODES)
        dst = _pad_edges(
            jnp.concatenate([edge_index_list[bi, 1], loop]), pad_e, NUM_NODES)

        deg_part = _deg_kernel(dst, ones_rows, zeros_init)
        degsum = deg_part[0] + deg_part[1]
        dinv = lax.rsqrt(degsum[:NUM_NODES, :1])

        xb = x[bi].T
        y1 = jnp.zeros((NPAD, D), jnp.float32).at[:NUM_NODES, :SEQ].set(xb * dinv)
        s1 = _agg_kernel(y1, src, dst, zeros_init)
        y2 = _mid_call(s1[0] + s1[1], degsum, W1, b1.reshape(1, HID), W2)
        s2 = _agg_kernel(y2, src, dst, zeros_init)

        degsums.append(degsum[:NUM_NODES])
        ssum2s.append((s2[0] + s2[1])[:NUM_NODES])

    out_flat = _conv_call(ssum2s[0], ssum2s[1], degsums[0], degsums[1],
                          b2.reshape(1, SEQ), Wc[:, :, 0].T, Wc[:, :, 1].T,
                          Wc[:, :, 2].T)
    return jnp.stack([out_flat[:SEQ], out_flat[SEQ:]], axis=0)


# speedup vs baseline: 38.0260x; 1.0015x over previous
import functools

import jax
import jax.numpy as jnp
from jax import lax
from jax.experimental import pallas as pl
from jax.experimental.pallas import tpu as pltpu
from jax.experimental.pallas import tpu_sc as plsc

NUM_NODES = 50000
SEQ = 12
HID = 64
NC = 2
NS = 16
NW = NC * NS
CHUNK = 128
CPW = 208
EPW = CHUNK * CPW
EPAD = EPW * NW
NPAD = 51200
D = 16
ROWS_PS = NPAD // NS

_mesh = plsc.VectorSubcoreMesh(core_axis_name="c", subcore_axis_name="s")


@functools.partial(
    pl.kernel,
    mesh=_mesh,
    out_type=jax.ShapeDtypeStruct((NC, NPAD, D), jnp.float32),
    compiler_params=pltpu.CompilerParams(use_tc_tiling_on_sc=False),
    scratch_types=[
        pltpu.VMEM((CPW, CHUNK), jnp.int32),
        pltpu.VMEM((CHUNK, D), jnp.float32),
        pltpu.VMEM_SHARED((NPAD, D), jnp.float32),
        pltpu.SemaphoreType.DMA,
        pltpu.SemaphoreType.DMA,
        pltpu.SemaphoreType.DMA,
        pltpu.SemaphoreType.DMA,
    ],
)
def _deg_kernel(dst_hbm, ones_hbm, zeros_hbm, out_hbm, idx_v, ones_v, acc,
                s0, s1, s2, s3):
    cid = lax.axis_index("c")
    sid = lax.axis_index("s")
    wid = sid * NC + cid
    sems = (s0, s1, s2, s3)
    pltpu.sync_copy(ones_hbm, ones_v)
    pltpu.sync_copy(zeros_hbm.at[pl.ds(sid * ROWS_PS, ROWS_PS)],
                    acc.at[pl.ds(sid * ROWS_PS, ROWS_PS)])
    pltpu.sync_copy(dst_hbm.at[pl.ds(wid * CPW, CPW)], idx_v)
    plsc.subcore_barrier()

    def body(g, carry):
        base = g * 4
        handles = [
            pltpu.async_copy(ones_v, acc.at[idx_v.at[base + b]], sems[b],
                             add=True)
            for b in range(4)
        ]
        for h in handles:
            h.wait()
        return carry

    lax.fori_loop(0, CPW // 4, body, 0)
    plsc.subcore_barrier()
    pltpu.sync_copy(acc.at[pl.ds(sid * ROWS_PS, ROWS_PS)],
                    out_hbm.at[cid, pl.ds(sid * ROWS_PS, ROWS_PS)])


@functools.partial(
    pl.kernel,
    mesh=_mesh,
    out_type=jax.ShapeDtypeStruct((NC, NPAD, D), jnp.float32),
    compiler_params=pltpu.CompilerParams(use_tc_tiling_on_sc=False),
    scratch_types=[
        pltpu.VMEM((CPW, CHUNK), jnp.int32),
        pltpu.VMEM((CPW, CHUNK), jnp.int32),
        pltpu.VMEM((4, CHUNK, D), jnp.float32),
        pltpu.VMEM_SHARED((NPAD, D), jnp.float32),
        pltpu.SemaphoreType.DMA,
        pltpu.SemaphoreType.DMA,
        pltpu.SemaphoreType.DMA,
        pltpu.SemaphoreType.DMA,
    ],
)
def _agg_kernel(y_hbm, src_hbm, dst_hbm, zeros_hbm, out_hbm,
                src_v, dst_v, rows_v, acc, s0, s1, s2, s3):
    cid = lax.axis_index("c")
    sid = lax.axis_index("s")
    wid = sid * NC + cid
    sems = (s0, s1, s2, s3)
    pltpu.sync_copy(zeros_hbm.at[pl.ds(sid * ROWS_PS, ROWS_PS)],
                    acc.at[pl.ds(sid * ROWS_PS, ROWS_PS)])
    pltpu.sync_copy(src_hbm.at[pl.ds(wid * CPW, CPW)], src_v)
    pltpu.sync_copy(dst_hbm.at[pl.ds(wid * CPW, CPW)], dst_v)
    plsc.subcore_barrier()

    def body(g, carry):
        base = g * 4
        handles = [
            pltpu.async_copy(y_hbm.at[src_v.at[base + b]], rows_v.at[b], sems[b])
            for b in range(4)
        ]
        for b in range(4):
            handles[b].wait()
            pltpu.sync_copy(rows_v.at[b], acc.at[dst_v.at[base + b]], add=True)
        return carry

    lax.fori_loop(0, CPW // 4, body, 0)
    plsc.subcore_barrier()
    pltpu.sync_copy(acc.at[pl.ds(sid * ROWS_PS, ROWS_PS)],
                    out_hbm.at[cid, pl.ds(sid * ROWS_PS, ROWS_PS)])


BLK_MID = 2048
NBLK_MID = NPAD // BLK_MID
BLK_CONV = 2000
NBLK_CONV = NUM_NODES // BLK_CONV


def _mid_body(s1_ref, deg_ref, w1_ref, b1_ref, w2_ref, y2_ref):
    dinv = lax.rsqrt(jnp.maximum(deg_ref[:, :1], 1.0))
    a1 = s1_ref[:, :SEQ] * dinv
    h1 = jnp.maximum(jnp.dot(a1, w1_ref[...],
                             preferred_element_type=jnp.float32) + b1_ref[...], 0.0)
    g = jnp.dot(h1, w2_ref[...], preferred_element_type=jnp.float32)
    y2_ref[:, :SEQ] = g * dinv
    y2_ref[:, SEQ:] = jnp.zeros((BLK_MID, D - SEQ), jnp.float32)


_mid_call = pl.pallas_call(
    _mid_body,
    grid=(NBLK_MID,),
    in_specs=[
        pl.BlockSpec((BLK_MID, D), lambda i: (i, 0)),
        pl.BlockSpec((BLK_MID, D), lambda i: (i, 0)),
        pl.BlockSpec((SEQ, HID), lambda i: (0, 0)),
        pl.BlockSpec((1, HID), lambda i: (0, 0)),
        pl.BlockSpec((HID, SEQ), lambda i: (0, 0)),
    ],
    out_specs=pl.BlockSpec((BLK_MID, D), lambda i: (i, 0)),
    out_shape=jax.ShapeDtypeStruct((NPAD, D), jnp.float32),
)


def _conv_body(s2a_ref, s2b_ref, dega_ref, degb_ref, b2_ref,
               w0_ref, w1_ref, w2_ref, out_ref):
    dinva = lax.rsqrt(jnp.maximum(dega_ref[:, :1], 1.0))
    dinvb = lax.rsqrt(jnp.maximum(degb_ref[:, :1], 1.0))
    hta = s2a_ref[:, :SEQ] * dinva + b2_ref[...]
    htb = s2b_ref[:, :SEQ] * dinvb + b2_ref[...]
    ht = jnp.concatenate([hta, htb], axis=1)
    dn = (((0,), (0,)), ((), ()))
    m0 = lax.dot_general(ht, w0_ref[...], dn,
                         preferred_element_type=jnp.float32)
    m1 = lax.dot_general(ht, w1_ref[...], dn,
                         preferred_element_type=jnp.float32)
    m2 = lax.dot_general(ht, w2_ref[...], dn,
                         preferred_element_type=jnp.float32)

    def roll1(a):
        return jnp.concatenate([a[-1:], a[:-1]], axis=0)

    def rollm1(a):
        return jnp.concatenate([a[1:], a[:1]], axis=0)

    acc_halves = []
    for h in range(2):
        sl = slice(h * SEQ, (h + 1) * SEQ)
        acc_halves.append(roll1(m0[sl]) + m1[sl] + rollm1(m2[sl]))
    acc = jnp.concatenate(acc_halves, axis=0)

    @pl.when(pl.program_id(0) == 0)
    def _init():
        out_ref[...] = jnp.zeros_like(out_ref)

    out_ref[...] += acc


_conv_call = pl.pallas_call(
    _conv_body,
    grid=(NBLK_CONV,),
    in_specs=[
        pl.BlockSpec((BLK_CONV, D), lambda i: (i, 0)),
        pl.BlockSpec((BLK_CONV, D), lambda i: (i, 0)),
        pl.BlockSpec((BLK_CONV, D), lambda i: (i, 0)),
        pl.BlockSpec((BLK_CONV, D), lambda i: (i, 0)),
        pl.BlockSpec((1, SEQ), lambda i: (0, 0)),
        pl.BlockSpec((BLK_CONV, HID), lambda i: (i, 0)),
        pl.BlockSpec((BLK_CONV, HID), lambda i: (i, 0)),
        pl.BlockSpec((BLK_CONV, HID), lambda i: (i, 0)),
    ],
    out_specs=pl.BlockSpec((2 * SEQ, HID), lambda i: (0, 0)),
    out_shape=jax.ShapeDtypeStruct((2 * SEQ, HID), jnp.float32),
)


def _pad_edges(idx, pad_e, fill):
    flat = jnp.concatenate(
        [idx, jnp.full((pad_e,), fill, jnp.int32)])
    return flat.reshape(EPAD // CHUNK, CHUNK)


def kernel(x, edge_index_list, W1, b1, W2, b2, Wc):
    batch = x.shape[0]
    n_edges = edge_index_list.shape[2]
    loop = jnp.arange(NUM_NODES, dtype=jnp.int32)
    pad_e = EPAD - (n_edges + NUM_NODES)
    zeros_init = jnp.zeros((NPAD, D), jnp.float32)
    ones_rows = jnp.ones((CHUNK, D), jnp.float32)

    degsums = []
    ssum2s = []
    for bi in range(batch):
        src = _pad_edges(
            jnp.concatenate([edge_index_list[bi, 0], loop]), pad_e, NUM_NODES)
        dst = _pad_edges(
            jnp.concatenate([edge_index_list[bi, 1], loop]), pad_e, NUM_NODES)

        deg_part = _deg_kernel(dst, ones_rows, zeros_init)
        degsum = deg_part[0] + deg_part[1]
        dinv = lax.rsqrt(degsum[:NUM_NODES, :1])

        xb = x[bi].T
        y1 = jnp.zeros((NPAD, D), jnp.float32).at[:NUM_NODES, :SEQ].set(xb * dinv)
        s1 = _agg_kernel(y1, src, dst, zeros_init)
        y2 = _mid_call(s1[0] + s1[1], degsum, W1, b1.reshape(1, HID), W2)
        s2 = _agg_kernel(y2, src, dst, zeros_init)

        degsums.append(degsum[:NUM_NODES])
        ssum2s.append((s2[0] + s2[1])[:NUM_NODES])

    out_flat = _conv_call(ssum2s[0], ssum2s[1], degsums[0], degsums[1],
                          b2.reshape(1, SEQ), Wc[:, :, 0].T, Wc[:, :, 1].T,
                          Wc[:, :, 2].T)
    return jnp.stack([out_flat[:SEQ], out_flat[SEQ:]], axis=0)
